# Initial kernel scaffold; baseline (speedup 1.0000x reference)
import jax
import jax.numpy as jnp
from jax.experimental import pallas as pl


def kernel(z, W):
    raise NotImplementedError("write your pallas kernel here")


# speedup vs baseline: 1.7794x; 1.7794x over previous
import functools

import jax
import jax.numpy as jnp
from jax import lax
from jax.experimental import pallas as pl
from jax.experimental.pallas import tpu as pltpu
from jax.experimental.pallas import tpu_sc as plsc

_N = 64 * 1024
_D = 64
_K = 512
_T = 2048
_G = _N // _T
_BETA = 0.25
_LOSS_SCALE = (1.0 + _BETA) / (_N * _D)

_NC, _NS = 2, 16
_NW = _NC * _NS
_TOK_PER_W = _N // _NW
_CH = 128
_NCHUNK = _TOK_PER_W // _CH


def _vq_tc_body(z_ref, w_ref, idx_ref, loss_ref):
    zb = z_ref[...]
    w = w_ref[...]
    scores = lax.dot_general(zb, w, (((1,), (1,)), ((), ())),
                             preferred_element_type=jnp.float32)
    wsq = jnp.sum(w * w, axis=1)
    zsq = jnp.sum(zb * zb, axis=1, keepdims=True)
    dist = (zsq - 2.0 * scores) + wsq[None, :]
    minval = jnp.min(dist, axis=1, keepdims=True)
    lane = lax.broadcasted_iota(jnp.int32, dist.shape, 1)
    idx = jnp.min(jnp.where(dist == minval, lane, _K), axis=1,
                  keepdims=True)
    idx_ref[...] = idx
    step = pl.program_id(0)
    prev = loss_ref[...]
    acc = jnp.where(step == 0, jnp.zeros_like(prev), prev) + jnp.sum(minval)
    loss_ref[...] = jnp.where(step == pl.num_programs(0) - 1,
                              acc * _LOSS_SCALE, acc)


_vq_tc = pl.pallas_call(
    _vq_tc_body,
    grid=(_G,),
    in_specs=[
        pl.BlockSpec((_T, _D), lambda i: (i, 0)),
        pl.BlockSpec((_K, _D), lambda i: (0, 0)),
    ],
    out_specs=[
        pl.BlockSpec((_T, 1), lambda i: (i, 0)),
        pl.BlockSpec((1, 1), lambda i: (0, 0)),
    ],
    out_shape=[
        jax.ShapeDtypeStruct((_N, 1), jnp.int32),
        jax.ShapeDtypeStruct((1, 1), jnp.float32),
    ],
)


def _sc_gather_body(idx_hbm, w_hbm, out_hbm, idx_v, rows_v, sem):
    wid = lax.axis_index("s") * _NC + lax.axis_index("c")
    pltpu.sync_copy(idx_hbm.at[pl.ds(wid * _NCHUNK, _NCHUNK)], idx_v)
    for g in range(_NCHUNK):
        pltpu.async_copy(w_hbm.at[idx_v.at[g]], rows_v, sem).wait()
        pltpu.sync_copy(
            rows_v, out_hbm.at[pl.ds(wid * _TOK_PER_W + g * _CH, _CH)])


@functools.cache
def _sc_gather():
    return pl.kernel(
        _sc_gather_body,
        mesh=plsc.VectorSubcoreMesh(core_axis_name="c", subcore_axis_name="s"),
        out_type=jax.ShapeDtypeStruct((_N, 128), jnp.float32),
        scratch_types=[
            pltpu.VMEM((_NCHUNK, _CH), jnp.int32),
            pltpu.VMEM((_CH, 128), jnp.float32),
            pltpu.SemaphoreType.DMA,
        ],
    )


def kernel(z, W):
    zf = z.reshape(_N, _D)
    idx_col, loss_arr = _vq_tc(zf, W)
    w_pad = jnp.pad(W, ((0, 0), (0, 128 - _D)))
    z_q = _sc_gather()(idx_col.reshape(_N // _CH, _CH), w_pad)[:, :_D]
    return (z_q.reshape(z.shape), loss_arr[0, 0],
            idx_col.reshape(z.shape[0], z.shape[1]))


# speedup vs baseline: 2.0018x; 1.1250x over previous
import functools

import jax
import jax.numpy as jnp
from jax import lax
from jax.experimental import pallas as pl
from jax.experimental.pallas import tpu as pltpu
from jax.experimental.pallas import tpu_sc as plsc

_N = 64 * 1024
_D = 64
_K = 512
_T = 2048
_G = _N // _T
_BETA = 0.25
_LOSS_SCALE = (1.0 + _BETA) / (_N * _D)

_NC, _NS = 2, 16
_NW = _NC * _NS
_TOK_PER_W = _N // _NW
_CH = 128
_NCHUNK = _TOK_PER_W // _CH
_NBUF = 4


def _vq_tc_body(z_ref, wm2_ref, idx_ref, loss_ref):
    zb = z_ref[...]
    wm2 = wm2_ref[...]
    scores = lax.dot_general(zb, wm2, (((1,), (1,)), ((), ())),
                             preferred_element_type=jnp.float32)
    wsq = jnp.sum(wm2 * wm2, axis=1) * 0.25
    zsq = jnp.sum(zb * zb, axis=1, keepdims=True)
    dist = (zsq + scores) + wsq[None, :]
    minval = jnp.min(dist, axis=1, keepdims=True)
    lane = lax.broadcasted_iota(jnp.int32, dist.shape, 1)
    idx = jnp.min(jnp.where(dist == minval, lane, _K), axis=1,
                  keepdims=True)
    idx_ref[...] = idx.reshape(_T // _CH, _CH)
    step = pl.program_id(0)
    prev = loss_ref[...]
    acc = jnp.where(step == 0, jnp.zeros_like(prev), prev) + jnp.sum(minval)
    loss_ref[...] = jnp.where(step == pl.num_programs(0) - 1,
                              acc * _LOSS_SCALE, acc)


_vq_tc = pl.pallas_call(
    _vq_tc_body,
    grid=(_G,),
    in_specs=[
        pl.BlockSpec((_T, _D), lambda i: (i, 0)),
        pl.BlockSpec((_K, _D), lambda i: (0, 0)),
    ],
    out_specs=[
        pl.BlockSpec((_T // _CH, _CH), lambda i: (i, 0)),
        pl.BlockSpec((1, 1), lambda i: (0, 0)),
    ],
    out_shape=[
        jax.ShapeDtypeStruct((_N // _CH, _CH), jnp.int32),
        jax.ShapeDtypeStruct((1, 1), jnp.float32),
    ],
)


def _sc_gather_body(idx_hbm, w_hbm, out_hbm, idx_v, rows_v, gsem, wsem):
    wid = lax.axis_index("s") * _NC + lax.axis_index("c")
    pltpu.sync_copy(idx_hbm.at[pl.ds(wid * _NCHUNK, _NCHUNK)], idx_v)

    def gather(g, b):
        return pltpu.async_copy(w_hbm.at[idx_v.at[g]], rows_v.at[b],
                                gsem.at[b])

    def write(g, b):
        return pltpu.async_copy(
            rows_v.at[b], out_hbm.at[pl.ds(wid * _TOK_PER_W + g * _CH, _CH)],
            wsem.at[b])

    gh = [None] * _NBUF
    wh = [None] * _NBUF
    for b in range(_NBUF - 1):
        gh[b] = gather(b, b)
    for g in range(_NCHUNK):
        b = g % _NBUF
        a = g + _NBUF - 1
        if a < _NCHUNK:
            ab = a % _NBUF
            if wh[ab] is not None:
                wh[ab].wait()
            gh[ab] = gather(a, ab)
        gh[b].wait()
        wh[b] = write(g, b)
    for b in range(_NBUF):
        if wh[b] is not None:
            wh[b].wait()


@functools.cache
def _sc_gather():
    return pl.kernel(
        _sc_gather_body,
        mesh=plsc.VectorSubcoreMesh(core_axis_name="c", subcore_axis_name="s"),
        out_type=jax.ShapeDtypeStruct((_N, 128), jnp.float32),
        scratch_types=[
            pltpu.VMEM((_NCHUNK, _CH), jnp.int32),
            pltpu.VMEM((_NBUF, _CH, 128), jnp.float32),
            pltpu.SemaphoreType.DMA((_NBUF,)),
            pltpu.SemaphoreType.DMA((_NBUF,)),
        ],
    )


def kernel(z, W):
    zf = z.reshape(_N, _D)
    w_m2 = W * (-2.0)
    idx_rows, loss_arr = _vq_tc(zf, w_m2)
    w_pad = jnp.pad(W, ((0, 0), (0, 128 - _D)))
    z_q = _sc_gather()(idx_rows, w_pad)[:, :_D]
    return (z_q.reshape(z.shape), loss_arr[0, 0],
            idx_rows.reshape(z.shape[0], z.shape[1]))


# speedup vs baseline: 2.0053x; 1.0018x over previous
import functools

import jax
import jax.numpy as jnp
from jax import lax
from jax.experimental import pallas as pl
from jax.experimental.pallas import tpu as pltpu
from jax.experimental.pallas import tpu_sc as plsc

_N = 64 * 1024
_D = 64
_K = 512
_T = 2048
_G = _N // _T
_BETA = 0.25
_LOSS_SCALE = (1.0 + _BETA) / (_N * _D)

_NC, _NS = 2, 16
_NW = _NC * _NS
_TOK_PER_W = _N // _NW
_CH = 128
_NCHUNK = _TOK_PER_W // _CH
_NBUF = 4


def _vq_tc_body(z_ref, wm2_ref, idx_ref, loss_ref):
    zb = z_ref[...]
    wm2 = wm2_ref[...]
    scores = lax.dot_general(zb, wm2, (((1,), (1,)), ((), ())),
                             preferred_element_type=jnp.float32)
    wsq = jnp.sum(wm2 * wm2, axis=1) * 0.25
    zsq = jnp.sum(zb * zb, axis=1, keepdims=True)
    dist = (zsq + scores) + wsq[None, :]
    minval = jnp.min(dist, axis=1, keepdims=True)
    lane = lax.broadcasted_iota(jnp.int32, dist.shape, 1)
    idx = jnp.min(jnp.where(dist == minval, lane, _K), axis=1,
                  keepdims=True)
    idx_ref[...] = idx.reshape(_T // _CH, _CH)
    step = pl.program_id(0)
    prev = loss_ref[...]
    acc = jnp.where(step == 0, jnp.zeros_like(prev), prev) + jnp.sum(minval)
    loss_ref[...] = jnp.where(step == pl.num_programs(0) - 1,
                              acc * _LOSS_SCALE, acc)


_vq_tc = pl.pallas_call(
    _vq_tc_body,
    grid=(_G,),
    in_specs=[
        pl.BlockSpec((_T, _D), lambda i: (i, 0)),
        pl.BlockSpec((_K, _D), lambda i: (0, 0)),
    ],
    out_specs=[
        pl.BlockSpec((_T // _CH, _CH), lambda i: (i, 0)),
        pl.BlockSpec((1, 1), lambda i: (0, 0)),
    ],
    out_shape=[
        jax.ShapeDtypeStruct((_N // _CH, _CH), jnp.int32),
        jax.ShapeDtypeStruct((1, 1), jnp.float32),
    ],
)


def _sc_gather_body(idx_hbm, w_hbm, out_hbm, idx_v, rows_v, gsem, wsem):
    wid = lax.axis_index("s") * _NC + lax.axis_index("c")
    pltpu.sync_copy(idx_hbm.at[pl.ds(wid * _NCHUNK, _NCHUNK)], idx_v)

    def gather(g, b):
        return pltpu.async_copy(w_hbm.at[idx_v.at[g]], rows_v.at[b],
                                gsem.at[b])

    def write(g, b):
        return pltpu.async_copy(
            rows_v.at[b], out_hbm.at[pl.ds(wid * _TOK_PER_W + g * _CH, _CH)],
            wsem.at[b])

    gh = [None] * _NBUF
    wh = [None] * _NBUF
    for b in range(_NBUF - 2):
        gh[b] = gather(b, b)
    for g in range(_NCHUNK):
        b = g % _NBUF
        a = g + _NBUF - 2
        if a < _NCHUNK:
            ab = a % _NBUF
            if wh[ab] is not None:
                wh[ab].wait()
            gh[ab] = gather(a, ab)
        gh[b].wait()
        wh[b] = write(g, b)
    for b in range(_NBUF):
        if wh[b] is not None:
            wh[b].wait()


@functools.cache
def _sc_gather():
    return pl.kernel(
        _sc_gather_body,
        mesh=plsc.VectorSubcoreMesh(core_axis_name="c", subcore_axis_name="s"),
        out_type=jax.ShapeDtypeStruct((_N, 128), jnp.float32),
        scratch_types=[
            pltpu.VMEM((_NCHUNK, _CH), jnp.int32),
            pltpu.VMEM((_NBUF, _CH, 128), jnp.float32),
            pltpu.SemaphoreType.DMA((_NBUF,)),
            pltpu.SemaphoreType.DMA((_NBUF,)),
        ],
    )


def kernel(z, W):
    zf = z.reshape(_N, _D)
    w_m2 = W * (-2.0)
    idx_rows, loss_arr = _vq_tc(zf, w_m2)
    w_pad = jnp.pad(W, ((0, 0), (0, 128 - _D)))
    z_q = _sc_gather()(idx_rows, w_pad)[:, :_D]
    return (z_q.reshape(z.shape), loss_arr[0, 0],
            idx_rows.reshape(z.shape[0], z.shape[1]))


# speedup vs baseline: 2.6512x; 1.3221x over previous
import functools

import jax
import jax.numpy as jnp
from jax import lax
from jax.experimental import pallas as pl
from jax.experimental.pallas import tpu as pltpu
from jax.experimental.pallas import tpu_sc as plsc

_N = 64 * 1024
_D = 64
_K = 512
_T = 2048
_G = _N // _T
_BETA = 0.25
_LOSS_SCALE = (1.0 + _BETA) / (_N * _D)

_NC, _NS = 2, 16
_NW = _NC * _NS
_TOK_PER_W = _N // _NW
_CH = 128
_NCHUNK = _TOK_PER_W // _CH
_SUPER = 4


def _vq_tc_body(z_ref, wm2_ref, idx_ref, loss_ref):
    zb = z_ref[...]
    wm2 = wm2_ref[...]
    scores = lax.dot_general(zb, wm2, (((1,), (1,)), ((), ())),
                             preferred_element_type=jnp.float32)
    wsq = jnp.sum(wm2 * wm2, axis=1) * 0.25
    zsq = jnp.sum(zb * zb, axis=1, keepdims=True)
    dist = (zsq + scores) + wsq[None, :]
    minval = jnp.min(dist, axis=1, keepdims=True)
    lane = lax.broadcasted_iota(jnp.int32, dist.shape, 1)
    idx = jnp.min(jnp.where(dist == minval, lane, _K), axis=1,
                  keepdims=True)
    idx_ref[...] = idx.reshape(_T // _CH, _CH)
    step = pl.program_id(0)
    prev = loss_ref[...]
    acc = jnp.where(step == 0, jnp.zeros_like(prev), prev) + jnp.sum(minval)
    loss_ref[...] = jnp.where(step == pl.num_programs(0) - 1,
                              acc * _LOSS_SCALE, acc)


_vq_tc = pl.pallas_call(
    _vq_tc_body,
    grid=(_G,),
    in_specs=[
        pl.BlockSpec((_T, _D), lambda i: (i, 0)),
        pl.BlockSpec((_K, _D), lambda i: (0, 0)),
    ],
    out_specs=[
        pl.BlockSpec((_T // _CH, _CH), lambda i: (i, 0)),
        pl.BlockSpec((1, 1), lambda i: (0, 0)),
    ],
    out_shape=[
        jax.ShapeDtypeStruct((_N // _CH, _CH), jnp.int32),
        jax.ShapeDtypeStruct((1, 1), jnp.float32),
    ],
)


def _sc_gather_body(idx_hbm, w_hbm, out_hbm, idx_v, w_sp, rows_v,
                    gsem, wsem):
    wid = lax.axis_index("s") * _NC + lax.axis_index("c")
    sid = lax.axis_index("s")

    @pl.when(sid == 0)
    def _():
        pltpu.sync_copy(w_hbm, w_sp)
    plsc.subcore_barrier()
    pltpu.sync_copy(idx_hbm.at[pl.ds(wid * _NCHUNK, _NCHUNK)], idx_v)

    wh = [None, None]
    for g in range(_NCHUNK):
        b = g % 2
        if wh[b] is not None:
            wh[b].wait()
        pltpu.async_copy(w_sp.at[idx_v.at[g]], rows_v.at[b], gsem).wait()
        wh[b] = pltpu.async_copy(
            rows_v.at[b],
            out_hbm.at[pl.ds(wid * _TOK_PER_W + g * _CH, _CH)],
            wsem.at[b])
    for b in range(2):
        if wh[b] is not None:
            wh[b].wait()


@functools.cache
def _sc_gather():
    return pl.kernel(
        _sc_gather_body,
        mesh=plsc.VectorSubcoreMesh(core_axis_name="c", subcore_axis_name="s"),
        out_type=jax.ShapeDtypeStruct((_N, 128), jnp.float32),
        scratch_types=[
            pltpu.VMEM((_NCHUNK, _CH), jnp.int32),
            pltpu.VMEM_SHARED((_K, 128), jnp.float32),
            pltpu.VMEM((2, _CH, 128), jnp.float32),
            pltpu.SemaphoreType.DMA,
            pltpu.SemaphoreType.DMA((2,)),
        ],
    )


def kernel(z, W):
    zf = z.reshape(_N, _D)
    w_m2 = W * (-2.0)
    idx_rows, loss_arr = _vq_tc(zf, w_m2)
    w_pad = jnp.pad(W, ((0, 0), (0, 128 - _D)))
    z_q = _sc_gather()(idx_rows, w_pad)[:, :_D]
    return (z_q.reshape(z.shape), loss_arr[0, 0],
            idx_rows.reshape(z.shape[0], z.shape[1]))
